# baseline (speedup 1.0000x reference)
import functools

import jax
import jax.numpy as jnp
from jax import lax
from jax.experimental import pallas as pl
from jax.experimental.pallas import tpu as pltpu
from jax.experimental.pallas import tpu_sc as plsc

_NS = 16
_NC = 2
_C = 256


def _scale2_body(h_ref, s_ref, ng_ref, nf_ref, hn_ref, sn_ref):
    hn_ref[...] = h_ref[...] * ng_ref[...]
    sn_ref[...] = s_ref[...] * nf_ref[...]


def _scale2(h, s, norm_g, norm_f, bm_rows):
    n, d = h.shape
    grid = (n // bm_rows,)
    row_spec = pl.BlockSpec((bm_rows, d), lambda i: (i, 0))
    nrm_spec = pl.BlockSpec((bm_rows, 1), lambda i: (i, 0))
    out_shape = jax.ShapeDtypeStruct((n, d), jnp.float32)
    return pl.pallas_call(
        _scale2_body,
        grid=grid,
        in_specs=[row_spec, row_spec, nrm_spec, nrm_spec],
        out_specs=[row_spec, row_spec],
        out_shape=[out_shape, out_shape],
    )(h, s, norm_g, norm_f)


def _sc_aggregate(hn, sn, edges_g, edges_f, zeros, npad, ept):
    d = hn.shape[1]
    rows_pt = npad // _NS
    mesh = plsc.VectorSubcoreMesh(core_axis_name="c", subcore_axis_name="s")
    out_t = jax.ShapeDtypeStruct((npad, d), jnp.float32)

    @functools.partial(
        pl.kernel,
        out_type=[out_t, out_t],
        mesh=mesh,
        scratch_types=[
            pltpu.VMEM((2 * _C,), jnp.int32),
            pltpu.VMEM((_C, d), jnp.float32),
            pltpu.VMEM_SHARED((npad, d), jnp.float32),
            pltpu.SemaphoreType.DMA,
        ],
    )
    def agg(hn_h, sn_h, edges_g_h, edges_f_h, zeros_h,
            outg_h, outf_h, idx, rows, acc, sem):
        c = lax.axis_index("c")
        s = lax.axis_index("s")
        r0 = s * rows_pt
        pltpu.sync_copy(zeros_h.at[pl.ds(r0, rows_pt)],
                        acc.at[pl.ds(r0, rows_pt)])
        plsc.subcore_barrier()

        ebase = s * ept

        ebase2 = 2 * ebase

        def edge_loop(table_h, edges_h):
            def body(i, carry):
                base2 = ebase2 + i * (2 * _C)
                pltpu.sync_copy(edges_h.at[pl.ds(base2, 2 * _C)], idx)
                pltpu.async_copy(table_h.at[idx.at[pl.ds(_C, _C)]], rows,
                                 sem).wait()
                pltpu.sync_copy(rows, acc.at[idx.at[pl.ds(0, _C)]],
                                add=True)
                return carry
            lax.fori_loop(0, ept // _C, body, 0)

        @pl.when(c == 0)
        def _():
            edge_loop(hn_h, edges_g_h)

        @pl.when(c == 1)
        def _():
            edge_loop(sn_h, edges_f_h)

        plsc.subcore_barrier()

        @pl.when(c == 0)
        def _():
            pltpu.sync_copy(acc.at[pl.ds(r0, rows_pt)],
                            outg_h.at[pl.ds(r0, rows_pt)])

        @pl.when(c == 1)
        def _():
            pltpu.sync_copy(acc.at[pl.ds(r0, rows_pt)],
                            outf_h.at[pl.ds(r0, rows_pt)])

    return agg(hn, sn, edges_g, edges_f, zeros)


def _fuse3_body(ag_ref, af_ref, m_ref, wh_ref, ws_ref, wm_ref,
                ng_ref, nf_ref, bias_ref, o_ref):
    hg = jnp.dot(ag_ref[...], wh_ref[...],
                 preferred_element_type=jnp.float32) * ng_ref[...]
    hf = jnp.dot(af_ref[...], ws_ref[...],
                 preferred_element_type=jnp.float32) * nf_ref[...]
    hm = jnp.dot(m_ref[...], wm_ref[...],
                 preferred_element_type=jnp.float32)
    o_ref[...] = jnp.maximum(hg + hf + hm + bias_ref[...], 0.0)


def _fuse3(ag, af, m, wh, ws, wm, norm_g, norm_f, bias, bm_rows):
    n, d = m.shape
    d_out = wh.shape[1]
    grid = (n // bm_rows,)
    row_spec = pl.BlockSpec((bm_rows, d), lambda i: (i, 0))
    out_spec = pl.BlockSpec((bm_rows, d_out), lambda i: (i, 0))
    w_spec = pl.BlockSpec((d, d_out), lambda i: (0, 0))
    nrm_spec = pl.BlockSpec((bm_rows, 1), lambda i: (i, 0))
    b_spec = pl.BlockSpec((1, d_out), lambda i: (0, 0))
    return pl.pallas_call(
        _fuse3_body,
        grid=grid,
        in_specs=[row_spec, row_spec, row_spec, w_spec, w_spec, w_spec,
                  nrm_spec, nrm_spec, b_spec],
        out_specs=out_spec,
        out_shape=jax.ShapeDtypeStruct((n, d_out), jnp.float32),
    )(ag, af, m, wh, ws, wm, norm_g, norm_f, bias)


def kernel(h, s, m, edge_index_g, edge_index_f, norm_g, norm_f,
           wh, ws, wm, bh, bs, bm):
    n, d = h.shape
    e = edge_index_g.shape[1]

    bm_rows = 2000 if n % 2000 == 0 else 400

    hn, sn = _scale2(h, s, norm_g, norm_f, bm_rows)

    ept = -(-e // (_NS * _C)) * _C
    epad = ept * _NS
    npad = -(-(n + 1) // (_NS * 8)) * (_NS * 8)
    pad = epad - e
    src_g = edge_index_g[0]
    dst_g = edge_index_g[1]
    src_f = edge_index_f[0]
    dst_f = edge_index_f[1]
    if pad:
        zpad = jnp.zeros((pad,), jnp.int32)
        dpad = jnp.full((pad,), n, jnp.int32)
        src_g = jnp.concatenate([src_g, zpad])
        dst_g = jnp.concatenate([dst_g, dpad])
        src_f = jnp.concatenate([src_f, zpad])
        dst_f = jnp.concatenate([dst_f, dpad])
    def _inter(src, dst):
        return jnp.stack([dst.reshape(-1, _C), src.reshape(-1, _C)],
                         axis=1).reshape(-1)
    edges_g = _inter(src_g, dst_g)
    edges_f = _inter(src_f, dst_f)
    zeros = jnp.zeros((npad, d), jnp.float32)

    agg_g, agg_f = _sc_aggregate(hn, sn, edges_g, edges_f,
                                 zeros, npad, ept)

    bias = (bh + bs + bm).reshape(1, wh.shape[1])
    return _fuse3(agg_g[:n], agg_f[:n], m, wh, ws, wm, norm_g, norm_f,
                  bias, bm_rows)


# speedup vs baseline: 1.0190x; 1.0190x over previous
import functools

import jax
import jax.numpy as jnp
from jax import lax
from jax.experimental import pallas as pl
from jax.experimental.pallas import tpu as pltpu
from jax.experimental.pallas import tpu_sc as plsc

_NS = 16
_NC = 2
_C = 256


def _scale2_body(h_ref, s_ref, ng_ref, nf_ref, hn_ref, sn_ref):
    hn_ref[...] = h_ref[...] * ng_ref[...]
    sn_ref[...] = s_ref[...] * nf_ref[...]


def _scale2(h, s, norm_g, norm_f, bm_rows):
    n, d = h.shape
    grid = (n // bm_rows,)
    row_spec = pl.BlockSpec((bm_rows, d), lambda i: (i, 0))
    nrm_spec = pl.BlockSpec((bm_rows, 1), lambda i: (i, 0))
    out_shape = jax.ShapeDtypeStruct((n, d), jnp.float32)
    return pl.pallas_call(
        _scale2_body,
        grid=grid,
        in_specs=[row_spec, row_spec, nrm_spec, nrm_spec],
        out_specs=[row_spec, row_spec],
        out_shape=[out_shape, out_shape],
    )(h, s, norm_g, norm_f)


def _sc_aggregate(hn, sn, edges_g, edges_f, npad, ept):
    d = hn.shape[1]
    rows_pt = npad // _NS
    mesh = plsc.VectorSubcoreMesh(core_axis_name="c", subcore_axis_name="s")
    out_t = jax.ShapeDtypeStruct((npad, d), jnp.float32)

    @functools.partial(
        pl.kernel,
        out_type=[out_t, out_t],
        mesh=mesh,
        scratch_types=[
            pltpu.VMEM((2 * _C,), jnp.int32),
            pltpu.VMEM((_C, d), jnp.float32),
            pltpu.VMEM_SHARED((npad, d), jnp.float32),
            pltpu.SemaphoreType.DMA,
        ],
    )
    def agg(hn_h, sn_h, edges_g_h, edges_f_h,
            outg_h, outf_h, idx, rows, acc, sem):
        c = lax.axis_index("c")
        s = lax.axis_index("s")
        r0 = s * rows_pt

        def zrow(i, carry):
            for j in range(d // 16):
                rows[i, pl.ds(16 * j, 16)] = jnp.zeros((16,), jnp.float32)
            return carry
        lax.fori_loop(0, _C, zrow, 0)
        off = 0
        while off < rows_pt:
            sz = min(_C, rows_pt - off)
            pltpu.sync_copy(rows.at[pl.ds(0, sz)],
                            acc.at[pl.ds(r0 + off, sz)])
            off += sz
        plsc.subcore_barrier()

        ebase = s * ept

        ebase2 = 2 * ebase

        def edge_loop(table_h, edges_h):
            def body(i, carry):
                base2 = ebase2 + i * (2 * _C)
                pltpu.sync_copy(edges_h.at[pl.ds(base2, 2 * _C)], idx)
                pltpu.async_copy(table_h.at[idx.at[pl.ds(_C, _C)]], rows,
                                 sem).wait()
                pltpu.sync_copy(rows, acc.at[idx.at[pl.ds(0, _C)]],
                                add=True)
                return carry
            lax.fori_loop(0, ept // _C, body, 0)

        @pl.when(c == 0)
        def _():
            edge_loop(hn_h, edges_g_h)

        @pl.when(c == 1)
        def _():
            edge_loop(sn_h, edges_f_h)

        plsc.subcore_barrier()

        @pl.when(c == 0)
        def _():
            pltpu.sync_copy(acc.at[pl.ds(r0, rows_pt)],
                            outg_h.at[pl.ds(r0, rows_pt)])

        @pl.when(c == 1)
        def _():
            pltpu.sync_copy(acc.at[pl.ds(r0, rows_pt)],
                            outf_h.at[pl.ds(r0, rows_pt)])

    return agg(hn, sn, edges_g, edges_f)


def _fuse3_body(ag_ref, af_ref, m_ref, wh_ref, ws_ref, wm_ref,
                ng_ref, nf_ref, bias_ref, o_ref):
    hg = jnp.dot(ag_ref[...], wh_ref[...],
                 preferred_element_type=jnp.float32) * ng_ref[...]
    hf = jnp.dot(af_ref[...], ws_ref[...],
                 preferred_element_type=jnp.float32) * nf_ref[...]
    hm = jnp.dot(m_ref[...], wm_ref[...],
                 preferred_element_type=jnp.float32)
    o_ref[...] = jnp.maximum(hg + hf + hm + bias_ref[...], 0.0)


def _fuse3(ag, af, m, wh, ws, wm, norm_g, norm_f, bias, bm_rows):
    n, d = m.shape
    d_out = wh.shape[1]
    grid = (n // bm_rows,)
    row_spec = pl.BlockSpec((bm_rows, d), lambda i: (i, 0))
    out_spec = pl.BlockSpec((bm_rows, d_out), lambda i: (i, 0))
    w_spec = pl.BlockSpec((d, d_out), lambda i: (0, 0))
    nrm_spec = pl.BlockSpec((bm_rows, 1), lambda i: (i, 0))
    b_spec = pl.BlockSpec((1, d_out), lambda i: (0, 0))
    return pl.pallas_call(
        _fuse3_body,
        grid=grid,
        in_specs=[row_spec, row_spec, row_spec, w_spec, w_spec, w_spec,
                  nrm_spec, nrm_spec, b_spec],
        out_specs=out_spec,
        out_shape=jax.ShapeDtypeStruct((n, d_out), jnp.float32),
    )(ag, af, m, wh, ws, wm, norm_g, norm_f, bias)


def kernel(h, s, m, edge_index_g, edge_index_f, norm_g, norm_f,
           wh, ws, wm, bh, bs, bm):
    n, d = h.shape
    e = edge_index_g.shape[1]

    bm_rows = 2000 if n % 2000 == 0 else 400

    hn, sn = _scale2(h, s, norm_g, norm_f, bm_rows)

    ept = -(-e // (_NS * _C)) * _C
    epad = ept * _NS
    npad = -(-(n + 1) // (_NS * 8)) * (_NS * 8)
    pad = epad - e
    src_g = edge_index_g[0]
    dst_g = edge_index_g[1]
    src_f = edge_index_f[0]
    dst_f = edge_index_f[1]
    if pad:
        zpad = jnp.zeros((pad,), jnp.int32)
        dpad = jnp.full((pad,), n, jnp.int32)
        src_g = jnp.concatenate([src_g, zpad])
        dst_g = jnp.concatenate([dst_g, dpad])
        src_f = jnp.concatenate([src_f, zpad])
        dst_f = jnp.concatenate([dst_f, dpad])
    def _inter(src, dst):
        return jnp.stack([dst.reshape(-1, _C), src.reshape(-1, _C)],
                         axis=1).reshape(-1)
    edges_g = _inter(src_g, dst_g)
    edges_f = _inter(src_f, dst_f)

    agg_g, agg_f = _sc_aggregate(hn, sn, edges_g, edges_f, npad, ept)

    bias = (bh + bs + bm).reshape(1, wh.shape[1])
    return _fuse3(agg_g[:n], agg_f[:n], m, wh, ws, wm, norm_g, norm_f,
                  bias, bm_rows)
